# baseline (speedup 1.0000x reference)
import jax
import jax.numpy as jnp
from jax.experimental import pallas as pl
from jax.experimental.pallas import tpu as pltpu

_QP = 127.0
_QN = -128.0
_EPS_CLAMP = 1e-5
_S_EPS = 2e-6


def _pick(n, prefs):
    for p in prefs:
        if n % p == 0:
            return p
    return n


def _wsum_kernel(w_ref, o_ref):
    @pl.when(pl.program_id(0) == 0)
    def _():
        o_ref[...] = jnp.zeros_like(o_ref)

    o_ref[...] += jnp.sum(jnp.abs(w_ref[...]), keepdims=True)


def _wquant_kernel(sw_ref, w_ref, o_ref):
    sw = sw_ref[0, 0]
    o_ref[...] = jnp.clip(jnp.round(w_ref[...] * sw), -1.0, 1.0).astype(jnp.bfloat16)


def _xquant_kernel(swq_ref, x_ref, q_ref, sc_ref):
    x = x_ref[...]
    amax = jnp.clip(jnp.max(jnp.abs(x), axis=-1, keepdims=True), _EPS_CLAMP, None)
    s_act = _QP / amax
    q_ref[...] = jnp.clip(jnp.round(x * s_act), _QN, _QP).astype(jnp.bfloat16)
    sc_ref[...] = (amax + _S_EPS) * swq_ref[0, 0]


def _mm_kernel(q_ref, t_ref, sc_ref, o_ref):
    acc = jax.lax.dot_general(
        q_ref[...],
        t_ref[...],
        dimension_numbers=(((1,), (1,)), ((), ())),
        preferred_element_type=jnp.float32,
    )
    o_ref[...] = acc * sc_ref[...]


def kernel(x, w):
    t_dim, k_dim = x.shape
    o_dim, _ = w.shape

    bw = _pick(o_dim, (344, 256, 128, 8))
    wsum = pl.pallas_call(
        _wsum_kernel,
        grid=(o_dim // bw,),
        in_specs=[pl.BlockSpec((bw, k_dim), lambda i: (i, 0))],
        out_specs=pl.BlockSpec((1, 1), lambda i: (0, 0)),
        out_shape=jax.ShapeDtypeStruct((1, 1), jnp.float32),
        compiler_params=pltpu.CompilerParams(dimension_semantics=("arbitrary",)),
    )(w)
    s_w = 1.0 / jnp.clip(wsum / (o_dim * k_dim), _EPS_CLAMP, None)

    tw = pl.pallas_call(
        _wquant_kernel,
        grid=(o_dim // bw,),
        in_specs=[
            pl.BlockSpec(memory_space=pltpu.SMEM),
            pl.BlockSpec((bw, k_dim), lambda i: (i, 0)),
        ],
        out_specs=pl.BlockSpec((bw, k_dim), lambda i: (i, 0)),
        out_shape=jax.ShapeDtypeStruct((o_dim, k_dim), jnp.bfloat16),
    )(s_w, w)

    swq = s_w / _QP
    bxm = _pick(t_dim, (512, 256, 8))
    q, sc = pl.pallas_call(
        _xquant_kernel,
        grid=(t_dim // bxm,),
        in_specs=[
            pl.BlockSpec(memory_space=pltpu.SMEM),
            pl.BlockSpec((bxm, k_dim), lambda i: (i, 0)),
        ],
        out_specs=[
            pl.BlockSpec((bxm, k_dim), lambda i: (i, 0)),
            pl.BlockSpec((bxm, 1), lambda i: (i, 0)),
        ],
        out_shape=[
            jax.ShapeDtypeStruct((t_dim, k_dim), jnp.bfloat16),
            jax.ShapeDtypeStruct((t_dim, 1), jnp.float32),
        ],
    )(swq, x)

    bm = _pick(t_dim, (1024, 512, 256, 8))
    bn = _pick(o_dim, (256, 128))
    out = pl.pallas_call(
        _mm_kernel,
        grid=(t_dim // bm, o_dim // bn),
        in_specs=[
            pl.BlockSpec((bm, k_dim), lambda i, j: (i, 0)),
            pl.BlockSpec((bn, k_dim), lambda i, j: (j, 0)),
            pl.BlockSpec((bm, 1), lambda i, j: (i, 0)),
        ],
        out_specs=pl.BlockSpec((bm, bn), lambda i, j: (i, j)),
        out_shape=jax.ShapeDtypeStruct((t_dim, o_dim), jnp.float32),
        compiler_params=pltpu.CompilerParams(
            dimension_semantics=("parallel", "arbitrary"),
        ),
    )(q, tw, sc)
    return out


# speedup vs baseline: 1.0339x; 1.0339x over previous
import jax
import jax.numpy as jnp
from jax.experimental import pallas as pl
from jax.experimental.pallas import tpu as pltpu

_QP = 127.0
_QN = -128.0
_EPS_CLAMP = 1e-5
_S_EPS = 2e-6


def _pick(n, prefs):
    for p in prefs:
        if n % p == 0:
            return p
    return n


def _wsum_kernel(w_ref, o_ref):
    @pl.when(pl.program_id(0) == 0)
    def _():
        o_ref[...] = jnp.zeros_like(o_ref)

    o_ref[...] += jnp.sum(jnp.abs(w_ref[...]), keepdims=True)


def _xquant_kernel(x_ref, q_ref, am_ref):
    x = x_ref[...]
    amax = jnp.clip(jnp.max(jnp.abs(x), axis=-1, keepdims=True), _EPS_CLAMP, None)
    s_act = _QP / amax
    q_ref[...] = jnp.clip(jnp.round(x * s_act), _QN, _QP).astype(jnp.bfloat16)
    am_ref[...] = amax


def _mm_kernel(swq_ref, q_ref, w_ref, am_ref, o_ref):
    sw = swq_ref[0, 0]
    tw = jnp.clip(jnp.round(w_ref[...] * (sw * _QP)), -1.0, 1.0).astype(jnp.bfloat16)
    acc = jax.lax.dot_general(
        q_ref[...],
        tw,
        dimension_numbers=(((1,), (1,)), ((), ())),
        preferred_element_type=jnp.float32,
    )
    o_ref[...] = acc * ((am_ref[...] + _S_EPS) * sw)


def kernel(x, w):
    t_dim, k_dim = x.shape
    o_dim, _ = w.shape

    bw = _pick(o_dim, (344, 256, 128, 8))
    wsum = pl.pallas_call(
        _wsum_kernel,
        grid=(o_dim // bw,),
        in_specs=[pl.BlockSpec((bw, k_dim), lambda i: (i, 0))],
        out_specs=pl.BlockSpec((1, 1), lambda i: (0, 0)),
        out_shape=jax.ShapeDtypeStruct((1, 1), jnp.float32),
        compiler_params=pltpu.CompilerParams(dimension_semantics=("arbitrary",)),
    )(w)
    s_w = 1.0 / jnp.clip(wsum / (o_dim * k_dim), _EPS_CLAMP, None)
    swq = s_w / _QP

    bxm = _pick(t_dim, (512, 256, 8))
    q, am = pl.pallas_call(
        _xquant_kernel,
        grid=(t_dim // bxm,),
        in_specs=[pl.BlockSpec((bxm, k_dim), lambda i: (i, 0))],
        out_specs=[
            pl.BlockSpec((bxm, k_dim), lambda i: (i, 0)),
            pl.BlockSpec((bxm, 1), lambda i: (i, 0)),
        ],
        out_shape=[
            jax.ShapeDtypeStruct((t_dim, k_dim), jnp.bfloat16),
            jax.ShapeDtypeStruct((t_dim, 1), jnp.float32),
        ],
    )(x)

    bm = _pick(t_dim, (1024, 512, 256, 8))
    bn = _pick(o_dim, (256, 128))
    out = pl.pallas_call(
        _mm_kernel,
        grid=(t_dim // bm, o_dim // bn),
        in_specs=[
            pl.BlockSpec(memory_space=pltpu.SMEM),
            pl.BlockSpec((bm, k_dim), lambda i, j: (i, 0)),
            pl.BlockSpec((bn, k_dim), lambda i, j: (j, 0)),
            pl.BlockSpec((bm, 1), lambda i, j: (i, 0)),
        ],
        out_specs=pl.BlockSpec((bm, bn), lambda i, j: (i, j)),
        out_shape=jax.ShapeDtypeStruct((t_dim, o_dim), jnp.float32),
        compiler_params=pltpu.CompilerParams(
            dimension_semantics=("parallel", "arbitrary"),
        ),
    )(swq, q, w, am)
    return out


# speedup vs baseline: 1.0976x; 1.0616x over previous
import jax
import jax.numpy as jnp
from jax.experimental import pallas as pl
from jax.experimental.pallas import tpu as pltpu

_QP = 127.0
_QN = -128.0
_EPS_CLAMP = 1e-5
_S_EPS = 2e-6


def _pick(n, prefs):
    for p in prefs:
        if n % p == 0:
            return p
    return n


def _wsum_kernel(w_ref, o_ref):
    @pl.when(pl.program_id(0) == 0)
    def _():
        o_ref[...] = jnp.zeros_like(o_ref)

    o_ref[...] += jnp.sum(jnp.abs(w_ref[...]), keepdims=True)


def _xquant_kernel(x_ref, q_ref, am_ref):
    x = x_ref[...]
    amax = jnp.clip(jnp.max(jnp.abs(x), axis=-1, keepdims=True), _EPS_CLAMP, None)
    s_act = _QP / amax
    q_ref[...] = jnp.clip(jnp.round(x * s_act), _QN, _QP).astype(jnp.bfloat16)
    am_ref[...] = amax


def _mm_kernel(swq_ref, q_ref, w_ref, am_ref, o_ref):
    sw = swq_ref[0, 0]
    tw = jnp.clip(jnp.round(w_ref[...] * (sw * _QP)), -1.0, 1.0).astype(jnp.bfloat16)
    acc = jax.lax.dot_general(
        q_ref[...],
        tw,
        dimension_numbers=(((1,), (1,)), ((), ())),
        preferred_element_type=jnp.float32,
    )
    o_ref[...] = acc * ((am_ref[...] + _S_EPS) * sw)


def kernel(x, w):
    t_dim, k_dim = x.shape
    o_dim, _ = w.shape

    bw = _pick(o_dim, (344, 256, 128, 8))
    wsum = pl.pallas_call(
        _wsum_kernel,
        grid=(o_dim // bw,),
        in_specs=[pl.BlockSpec((bw, k_dim), lambda i: (i, 0))],
        out_specs=pl.BlockSpec((1, 1), lambda i: (0, 0)),
        out_shape=jax.ShapeDtypeStruct((1, 1), jnp.float32),
        compiler_params=pltpu.CompilerParams(dimension_semantics=("arbitrary",)),
    )(w)
    s_w = 1.0 / jnp.clip(wsum / (o_dim * k_dim), _EPS_CLAMP, None)
    swq = s_w / _QP

    bxm = _pick(t_dim, (512, 256, 8))
    q, am = pl.pallas_call(
        _xquant_kernel,
        grid=(t_dim // bxm,),
        in_specs=[pl.BlockSpec((bxm, k_dim), lambda i: (i, 0))],
        out_specs=[
            pl.BlockSpec((bxm, k_dim), lambda i: (i, 0)),
            pl.BlockSpec((bxm, 1), lambda i: (i, 0)),
        ],
        out_shape=[
            jax.ShapeDtypeStruct((t_dim, k_dim), jnp.bfloat16),
            jax.ShapeDtypeStruct((t_dim, 1), jnp.float32),
        ],
    )(x)

    bm = _pick(t_dim, (2048, 1024, 512, 256, 8))
    bn = _pick(o_dim, (256, 128))
    out = pl.pallas_call(
        _mm_kernel,
        grid=(t_dim // bm, o_dim // bn),
        in_specs=[
            pl.BlockSpec(memory_space=pltpu.SMEM),
            pl.BlockSpec((bm, k_dim), lambda i, j: (i, 0)),
            pl.BlockSpec((bn, k_dim), lambda i, j: (j, 0)),
            pl.BlockSpec((bm, 1), lambda i, j: (i, 0)),
        ],
        out_specs=pl.BlockSpec((bm, bn), lambda i, j: (i, j)),
        out_shape=jax.ShapeDtypeStruct((t_dim, o_dim), jnp.float32),
        compiler_params=pltpu.CompilerParams(
            dimension_semantics=("parallel", "arbitrary"),
            vmem_limit_bytes=58 * 1024 * 1024,
        ),
    )(swq, q, w, am)
    return out


# speedup vs baseline: 1.1278x; 1.0275x over previous
import jax
import jax.numpy as jnp
from jax.experimental import pallas as pl
from jax.experimental.pallas import tpu as pltpu

_QP = 127.0
_QN = -128.0
_EPS_CLAMP = 1e-5
_S_EPS = 2e-6


def _pick(n, prefs):
    for p in prefs:
        if n % p == 0:
            return p
    return n


def _prep_kernel(w_ref, x_ref, ws_ref, q_ref, am_ref):
    @pl.when(pl.program_id(0) == 0)
    def _():
        ws_ref[...] = jnp.zeros_like(ws_ref)

    ws_ref[...] += jnp.sum(jnp.abs(w_ref[...]), keepdims=True)
    x = x_ref[...]
    amax = jnp.clip(jnp.max(jnp.abs(x), axis=-1, keepdims=True), _EPS_CLAMP, None)
    s_act = _QP / amax
    q_ref[...] = jnp.clip(jnp.round(x * s_act), _QN, _QP).astype(jnp.bfloat16)
    am_ref[...] = amax


def _mm_kernel(swq_ref, q_ref, w_ref, am_ref, o_ref):
    sw = swq_ref[0, 0]
    tw = jnp.clip(jnp.round(w_ref[...] * (sw * _QP)), -1.0, 1.0).astype(jnp.bfloat16)
    acc = jax.lax.dot_general(
        q_ref[...],
        tw,
        dimension_numbers=(((1,), (1,)), ((), ())),
        preferred_element_type=jnp.float32,
    )
    o_ref[...] = acc * ((am_ref[...] + _S_EPS) * sw)


def kernel(x, w):
    t_dim, k_dim = x.shape
    o_dim, _ = w.shape

    g = 1
    for cand in (32, 16, 8, 4, 2):
        if o_dim % cand == 0 and t_dim % cand == 0 \
                and (o_dim // cand) % 8 == 0 and (t_dim // cand) % 8 == 0:
            g = cand
            break
    bw = o_dim // g
    bxm = t_dim // g
    wsum, q, am = pl.pallas_call(
        _prep_kernel,
        grid=(g,),
        in_specs=[
            pl.BlockSpec((bw, k_dim), lambda i: (i, 0)),
            pl.BlockSpec((bxm, k_dim), lambda i: (i, 0)),
        ],
        out_specs=[
            pl.BlockSpec((1, 1), lambda i: (0, 0)),
            pl.BlockSpec((bxm, k_dim), lambda i: (i, 0)),
            pl.BlockSpec((bxm, 1), lambda i: (i, 0)),
        ],
        out_shape=[
            jax.ShapeDtypeStruct((1, 1), jnp.float32),
            jax.ShapeDtypeStruct((t_dim, k_dim), jnp.bfloat16),
            jax.ShapeDtypeStruct((t_dim, 1), jnp.float32),
        ],
        compiler_params=pltpu.CompilerParams(dimension_semantics=("arbitrary",)),
    )(w, x)
    s_w = 1.0 / jnp.clip(wsum / (o_dim * k_dim), _EPS_CLAMP, None)
    swq = s_w / _QP

    bm = _pick(t_dim, (2048, 1024, 512, 256, 8))
    bn = _pick(o_dim, (256, 128))
    out = pl.pallas_call(
        _mm_kernel,
        grid=(t_dim // bm, o_dim // bn),
        in_specs=[
            pl.BlockSpec(memory_space=pltpu.SMEM),
            pl.BlockSpec((bm, k_dim), lambda i, j: (i, 0)),
            pl.BlockSpec((bn, k_dim), lambda i, j: (j, 0)),
            pl.BlockSpec((bm, 1), lambda i, j: (i, 0)),
        ],
        out_specs=pl.BlockSpec((bm, bn), lambda i, j: (i, j)),
        out_shape=jax.ShapeDtypeStruct((t_dim, o_dim), jnp.float32),
        compiler_params=pltpu.CompilerParams(
            dimension_semantics=("parallel", "arbitrary"),
            vmem_limit_bytes=58 * 1024 * 1024,
        ),
    )(swq, q, w, am)
    return out
